# Initial kernel scaffold; baseline (speedup 1.0000x reference)
import functools

import jax
import jax.numpy as jnp
from jax import lax
from jax.experimental import pallas as pl
from jax.experimental.pallas import tpu as pltpu
from jax.experimental.pallas import tpu_sc as plsc

NUM_USERS = 25000
N_NODES = 50000
D = 64
BATCH = 16384
N_EDGES = 800000

NC = 2
NS = 16
HALF = 25088
NPAD = NC * HALF
DUMMY = 1024
Q = HALF // NS
ZR = 112
C = 128
NCHUNK = N_EDGES // C
CHUNK_ITERS = -(-NCHUNK // NS)

_mesh = plsc.VectorSubcoreMesh(core_axis_name="c", subcore_axis_name="s")


def _layer_body(scale, table, src2, dst2, val2, rsum_in,
                new_table, rsum_out,
                acc, idx_v, dstl_v, val_v, rows_v, av, bv, zv, gsem):
  c = lax.axis_index("c")
  s = lax.axis_index("s")
  base = c * HALF

  for r in range(ZR):
    for q in range(D // 16):
      zv[r, pl.ds(q * 16, 16)] = jnp.zeros((16,), jnp.float32)

  def _zero(i, carry):
    pltpu.sync_copy(zv, acc.at[pl.ds(s * Q + i * ZR, ZR)])
    return carry
  lax.fori_loop(0, Q // ZR, _zero, 0)
  plsc.subcore_barrier()

  lanes = lax.iota(jnp.int32, 16)

  def _chunk(i, carry):
    k = i * NS + s

    @pl.when(k < NCHUNK)
    def _():
      pltpu.sync_copy(src2.at[k], idx_v)
      pltpu.sync_copy(dst2.at[k], dstl_v)
      pltpu.sync_copy(val2.at[k], val_v)
      pltpu.async_copy(table.at[idx_v], rows_v, gsem).wait()
      for g in range(C // 16):
        sl = pl.ds(g * 16, 16)
        loc = dstl_v[sl] - base
        ok = (loc >= 0) & (loc < HALF)
        spread = HALF + ((g * 16 + k + lanes) & (DUMMY - 1))
        dstl_v[sl] = jnp.where(ok, loc, spread)
        for j in range(16):
          e = g * 16 + j
          vs = plsc.load_gather(val_v, [jnp.full((16,), e, jnp.int32)])
          for q in range(D // 16):
            qs = pl.ds(q * 16, 16)
            rows_v[e, qs] = rows_v[e, qs] * vs
      pltpu.sync_copy(rows_v, acc.at[dstl_v], add=True)
    return carry
  lax.fori_loop(0, CHUNK_ITERS, _chunk, 0)
  plsc.subcore_barrier()

  pltpu.sync_copy(acc.at[pl.ds(s * Q, Q)],
                  new_table.at[pl.ds(base + s * Q, Q)])

  def _rsum(i, carry):
    r0 = s * Q + i * ZR
    pltpu.sync_copy(rsum_in.at[pl.ds(base + r0, ZR)], av)
    pltpu.sync_copy(acc.at[pl.ds(r0, ZR)], bv)
    for r in range(ZR):
      for q in range(D // 16):
        qs = pl.ds(q * 16, 16)
        av[r, qs] = (av[r, qs] + bv[r, qs]) * scale
    pltpu.sync_copy(av, rsum_out.at[pl.ds(base + r0, ZR)])
    return carry
  lax.fori_loop(0, Q // ZR, _rsum, 0)


def _make_layer(scale):
  return pl.kernel(
      functools.partial(_layer_body, scale),
      out_type=(
          jax.ShapeDtypeStruct((NPAD, D), jnp.float32),
          jax.ShapeDtypeStruct((NPAD, D), jnp.float32),
      ),
      mesh=_mesh,
      scratch_types=[
          pltpu.VMEM_SHARED((HALF + DUMMY, D), jnp.float32),
          pltpu.VMEM((C,), jnp.int32),
          pltpu.VMEM((C,), jnp.int32),
          pltpu.VMEM((C,), jnp.float32),
          pltpu.VMEM((C, D), jnp.float32),
          pltpu.VMEM((ZR, D), jnp.float32),
          pltpu.VMEM((ZR, D), jnp.float32),
          pltpu.VMEM((ZR, D), jnp.float32),
          pltpu.SemaphoreType.DMA,
      ],
  )


def _gather_body(rsum, uidx2, iidx2, out_u, out_i, idx_v, rows_v, gsem):
  c = lax.axis_index("c")
  s = lax.axis_index("s")
  w = s * NC + c

  def _do(idx2, out, offset, j, carry):
    r = w * 4 + j
    pltpu.sync_copy(idx2.at[r], idx_v)
    if offset:
      for g in range(C // 16):
        sl = pl.ds(g * 16, 16)
        idx_v[sl] = idx_v[sl] + offset
    pltpu.async_copy(rsum.at[idx_v], rows_v, gsem).wait()
    pltpu.sync_copy(rows_v, out.at[pl.ds(r * C, C)])
    return carry

  lax.fori_loop(0, 4, functools.partial(_do, uidx2, out_u, 0), 0)
  lax.fori_loop(0, 4, functools.partial(_do, iidx2, out_i, NUM_USERS), 0)


_gather_kernel = pl.kernel(
    _gather_body,
    out_type=(
        jax.ShapeDtypeStruct((BATCH, D), jnp.float32),
        jax.ShapeDtypeStruct((BATCH, D), jnp.float32),
    ),
    mesh=_mesh,
    scratch_types=[
        pltpu.VMEM((C,), jnp.int32),
        pltpu.VMEM((C, D), jnp.float32),
        pltpu.SemaphoreType.DMA,
    ],
)


def kernel(users, items, edge_index, edge_vals, user_emb, item_emb):
  src2 = edge_index[0].reshape(NCHUNK, C)
  dst2 = edge_index[1].reshape(NCHUNK, C)
  val2 = edge_vals.reshape(NCHUNK, C)
  emb0 = jnp.concatenate(
      [user_emb, item_emb,
       jnp.zeros((NPAD - N_NODES, D), jnp.float32)], axis=0)

  table, rsum = emb0, emb0
  layer1 = _make_layer(1.0)
  layer_last = _make_layer(0.25)
  table, rsum = layer1(table, src2, dst2, val2, rsum)
  table, rsum = layer1(table, src2, dst2, val2, rsum)
  table, rsum = layer_last(table, src2, dst2, val2, rsum)

  uidx2 = users.reshape(BATCH // C, C)
  iidx2 = items.reshape(BATCH // C, C)
  return _gather_kernel(rsum, uidx2, iidx2)


# speedup vs baseline: 2.9344x; 2.9344x over previous
import functools

import jax
import jax.numpy as jnp
from jax import lax
from jax.experimental import pallas as pl
from jax.experimental.pallas import tpu as pltpu
from jax.experimental.pallas import tpu_sc as plsc

NUM_USERS = 25000
N_NODES = 50000
D = 64
BATCH = 16384
N_EDGES = 800000

NC = 2
NS = 16
HALF = 25088
NPAD = NC * HALF
DUMMY = 512
Q = HALF // NS
ZR = 56

SUBC = 64
NQ = 4
ESUP = SUBC * NQ
NSUPER = N_EDGES // ESUP
JMAX = -(-NSUPER // NS)
PAIRS = (JMAX + 1) // 2

GC = 128

_mesh = plsc.VectorSubcoreMesh(core_axis_name="c", subcore_axis_name="s")


def _layer_body(scale, table, src3, dst3, valf, rsum_in,
                new_table, rsum_out,
                acc, src_st, dstl_st, val_st,
                rows0, rows1, rows2, rows3,
                g0, g1, g2, g3, s0, s1, s2, s3, l0, l1):
  c = lax.axis_index("c")
  s = lax.axis_index("s")
  base = c * HALF
  rows = (rows0, rows1, rows2, rows3)
  gsems = (g0, g1, g2, g3)
  ssems = (s0, s1, s2, s3)
  lsems = (l0, l1)
  lanes = lax.iota(jnp.int32, 16)

  for r in range(ZR):
    for q in range(D // 16):
      rows0[r, pl.ds(q * 16, 16)] = jnp.zeros((16,), jnp.float32)

  def _zero(i, carry):
    pltpu.sync_copy(rows0.at[pl.ds(0, ZR)], acc.at[pl.ds(s * Q + i * ZR, ZR)])
    return carry
  lax.fori_loop(0, Q // ZR, _zero, 0)
  plsc.subcore_barrier()

  def _stage_issue(p, k):
    pltpu.async_copy(src3.at[k], src_st.at[p], lsems[p])
    pltpu.async_copy(dst3.at[k], dstl_st.at[p], lsems[p])
    pltpu.async_copy(valf.at[pl.ds(k * ESUP, ESUP)],
                     val_st.at[p, pl.ds(16, ESUP)], lsems[p])

  def _stage_wait(p, k):
    pltpu.make_async_copy(src3.at[k], src_st.at[p], lsems[p]).wait()
    pltpu.make_async_copy(dst3.at[k], dstl_st.at[p], lsems[p]).wait()
    pltpu.make_async_copy(valf.at[pl.ds(k * ESUP, ESUP)],
                          val_st.at[p, pl.ds(16, ESUP)], lsems[p]).wait()

  def _wait_scatters(p):
    for r in range(NQ):
      pltpu.make_async_copy(rows[r], acc.at[dstl_st.at[p, r]],
                            ssems[r]).wait()

  def _super(p, k):
    for r in range(NQ):
      pltpu.async_copy(table.at[src_st.at[p, r]], rows[r], gsems[r])
    for r in range(NQ):
      pltpu.make_async_copy(table.at[src_st.at[p, r]], rows[r],
                            gsems[r]).wait()
      dref = dstl_st.at[p, r]
      for g in range(SUBC // 16):
        sl = pl.ds(g * 16, 16)
        loc = dref[sl] - base
        ok = (loc >= 0) & (loc < HALF)
        spread = HALF + ((g * 16 + k + lanes) & (DUMMY - 1))
        dref[sl] = jnp.where(ok, loc, spread)
      vref = val_st.at[p]
      for j in range(SUBC):
        vs = plsc.load_gather(
            vref, [jnp.full((16,), 16 + r * SUBC + j, jnp.int32)])
        for q in range(D // 16):
          qs = pl.ds(q * 16, 16)
          rows[r][j, qs] = rows[r][j, qs] * vs
      pltpu.async_copy(rows[r], acc.at[dstl_st.at[p, r]], ssems[r],
                       add=True)

  @pl.when(s < NSUPER)
  def _():
    _stage_issue(0, s)

  def _pair(i, carry):
    for p in (0, 1):
      j = 2 * i + p
      k = j * NS + s

      @pl.when((k >= NS) & (k - NS < NSUPER))
      def _(p=p, k=k):
        _wait_scatters(1 - p)

      @pl.when(k + NS < NSUPER)
      def _(p=p, k=k):
        _stage_issue(1 - p, k + NS)

      @pl.when(k < NSUPER)
      def _(p=p, k=k):
        _stage_wait(p, k)

      @pl.when(k < NSUPER)
      def _(p=p, k=k):
        _super(p, k)
    return carry
  lax.fori_loop(0, PAIRS, _pair, 0)

  k_ep = 2 * PAIRS * NS + s

  @pl.when(k_ep - NS < NSUPER)
  def _():
    _wait_scatters(1)

  plsc.subcore_barrier()

  pltpu.sync_copy(acc.at[pl.ds(s * Q, Q)],
                  new_table.at[pl.ds(base + s * Q, Q)])

  def _rsum(i, carry):
    r0 = s * Q + i * ZR
    pltpu.sync_copy(rsum_in.at[pl.ds(base + r0, ZR)], rows0.at[pl.ds(0, ZR)])
    pltpu.sync_copy(acc.at[pl.ds(r0, ZR)], rows1.at[pl.ds(0, ZR)])
    for r in range(ZR):
      for q in range(D // 16):
        qs = pl.ds(q * 16, 16)
        rows0[r, qs] = (rows0[r, qs] + rows1[r, qs]) * scale
    pltpu.sync_copy(rows0.at[pl.ds(0, ZR)],
                    rsum_out.at[pl.ds(base + r0, ZR)])
    return carry
  lax.fori_loop(0, Q // ZR, _rsum, 0)


def _make_layer(scale):
  return pl.kernel(
      functools.partial(_layer_body, scale),
      out_type=(
          jax.ShapeDtypeStruct((NPAD, D), jnp.float32),
          jax.ShapeDtypeStruct((NPAD, D), jnp.float32),
      ),
      mesh=_mesh,
      compiler_params=pltpu.CompilerParams(
          needs_layout_passes=False, use_tc_tiling_on_sc=False),
      scratch_types=[
          pltpu.VMEM_SHARED((HALF + DUMMY, D), jnp.float32),
          pltpu.VMEM((2, NQ, SUBC), jnp.int32),
          pltpu.VMEM((2, NQ, SUBC), jnp.int32),
          pltpu.VMEM((2, ESUP + 16), jnp.float32),
          pltpu.VMEM((SUBC, D), jnp.float32),
          pltpu.VMEM((SUBC, D), jnp.float32),
          pltpu.VMEM((SUBC, D), jnp.float32),
          pltpu.VMEM((SUBC, D), jnp.float32),
          pltpu.SemaphoreType.DMA,
          pltpu.SemaphoreType.DMA,
          pltpu.SemaphoreType.DMA,
          pltpu.SemaphoreType.DMA,
          pltpu.SemaphoreType.DMA,
          pltpu.SemaphoreType.DMA,
          pltpu.SemaphoreType.DMA,
          pltpu.SemaphoreType.DMA,
          pltpu.SemaphoreType.DMA,
          pltpu.SemaphoreType.DMA,
      ],
  )


def _gather_body(rsum, uidx2, iidx2, out_u, out_i, idx_v, rows_v, gsem):
  c = lax.axis_index("c")
  s = lax.axis_index("s")
  w = s * NC + c

  def _do(idx2, out, offset, j, carry):
    r = w * 4 + j
    pltpu.sync_copy(idx2.at[r], idx_v)
    if offset:
      for g in range(GC // 16):
        sl = pl.ds(g * 16, 16)
        idx_v[sl] = idx_v[sl] + offset
    pltpu.async_copy(rsum.at[idx_v], rows_v, gsem).wait()
    pltpu.sync_copy(rows_v, out.at[pl.ds(r * GC, GC)])
    return carry

  lax.fori_loop(0, 4, functools.partial(_do, uidx2, out_u, 0), 0)
  lax.fori_loop(0, 4, functools.partial(_do, iidx2, out_i, NUM_USERS), 0)


_gather_kernel = pl.kernel(
    _gather_body,
    out_type=(
        jax.ShapeDtypeStruct((BATCH, D), jnp.float32),
        jax.ShapeDtypeStruct((BATCH, D), jnp.float32),
    ),
    mesh=_mesh,
    compiler_params=pltpu.CompilerParams(
        needs_layout_passes=False, use_tc_tiling_on_sc=False),
    scratch_types=[
        pltpu.VMEM((GC,), jnp.int32),
        pltpu.VMEM((GC, D), jnp.float32),
        pltpu.SemaphoreType.DMA,
    ],
)


def kernel(users, items, edge_index, edge_vals, user_emb, item_emb):
  src3 = edge_index[0].reshape(NSUPER, NQ, SUBC)
  dst3 = edge_index[1].reshape(NSUPER, NQ, SUBC)
  emb0 = jnp.concatenate(
      [user_emb, item_emb,
       jnp.zeros((NPAD - N_NODES, D), jnp.float32)], axis=0)

  table, rsum = emb0, emb0
  layer1 = _make_layer(1.0)
  layer_last = _make_layer(0.25)
  table, rsum = layer1(table, src3, dst3, edge_vals, rsum)
  table, rsum = layer1(table, src3, dst3, edge_vals, rsum)
  table, rsum = layer_last(table, src3, dst3, edge_vals, rsum)

  uidx2 = users.reshape(BATCH // GC, GC)
  iidx2 = items.reshape(BATCH // GC, GC)
  return _gather_kernel(rsum, uidx2, iidx2)
